# baseline (speedup 1.0000x reference)
import math

import numpy as np
import jax
import jax.numpy as jnp
from jax.experimental import pallas as pl

_E = 64
_TOPK = int(math.log(_E))


def _dft_mats():
    e = np.arange(_E)
    phase = 2.0 * np.pi * np.outer(e, e) / _E
    c = np.cos(phase)
    s = -np.sin(phase)
    fwd = np.concatenate([c, s], axis=1).astype(np.float32)
    inv = np.concatenate([c, s], axis=0).astype(np.float32)
    return fwd, inv


_FWD, _INV = _dft_mats()


def _dot(a, b):
    return jax.lax.dot_general(
        a, b, (((1,), (0,)), ((), ())),
        preferred_element_type=jnp.float32,
        precision=jax.lax.Precision.HIGHEST)


def _body(q_ref, k_ref, v_ref, fwd_ref, inv_ref, v_out_ref, corr_out_ref):
    q = q_ref[0, 0]
    k = k_ref[0, 0]
    v = v_ref[0, 0]
    fwd = fwd_ref[...]
    inv = inv_ref[...]
    r = q.shape[0]

    qkf = _dot(jnp.concatenate([q, k], axis=0), fwd)
    qr, qi = qkf[:r, :_E], qkf[:r, _E:]
    kr, ki = qkf[r:, :_E], qkf[r:, _E:]
    pr = qr * kr + qi * ki
    pi = qi * kr - qr * ki
    corr = _dot(jnp.concatenate([pr, pi], axis=1), inv) * (1.0 / _E)
    corr_out_ref[0] = corr.T

    neg = jnp.float32(-jnp.inf)
    m1 = jnp.max(corr, axis=1, keepdims=True)
    work = jnp.where(corr == m1, neg, corr)
    m2 = jnp.max(work, axis=1, keepdims=True)
    work = jnp.where(work == m2, neg, work)
    m3 = jnp.max(work, axis=1, keepdims=True)
    work = jnp.where(work == m3, neg, work)
    m4 = jnp.max(work, axis=1, keepdims=True)

    num = jnp.where(corr >= m4, jnp.exp(corr - m1), 0.0)
    denom = jnp.sum(num, axis=1, keepdims=True)
    w = num * (1.0 / denom)

    wvf = _dot(jnp.concatenate([w, v], axis=0), fwd)
    wr, wi = wvf[:r, :_E], wvf[:r, _E:]
    vr, vi = wvf[r:, :_E], wvf[r:, _E:]
    gr = wr * vr + wi * vi
    gi = wr * vi - wi * vr
    v_out_ref[0, 0] = _dot(jnp.concatenate([gr, gi], axis=1), inv) * (1.0 / _E)


def kernel(queries, keys, values):
    B, H, L, E = queries.shape
    lblk = 512
    grid = (B, H, L // lblk)
    row_spec = pl.BlockSpec((1, 1, lblk, E), lambda b, h, l: (b, h, l, 0))
    out_v, out_corr = pl.pallas_call(
        _body,
        grid=grid,
        in_specs=[
            row_spec, row_spec, row_spec,
            pl.BlockSpec((_E, 2 * _E), lambda b, h, l: (0, 0)),
            pl.BlockSpec((2 * _E, _E), lambda b, h, l: (0, 0)),
        ],
        out_specs=[
            row_spec,
            pl.BlockSpec((1, E, lblk), lambda b, h, l: (b, 0, h * (L // lblk) + l)),
        ],
        out_shape=[
            jax.ShapeDtypeStruct((B, H, L, E), jnp.float32),
            jax.ShapeDtypeStruct((B, E, H * L), jnp.float32),
        ],
    )(queries, keys, values, jnp.asarray(_FWD), jnp.asarray(_INV))
    return (out_v, out_corr.reshape(B, E, H, L))


# speedup vs baseline: 1.1330x; 1.1330x over previous
import math

import numpy as np
import jax
import jax.numpy as jnp
from jax.experimental import pallas as pl

_E = 64
_TOPK = int(math.log(_E))


def _dft_mats():
    e = np.arange(_E)
    phase = 2.0 * np.pi * np.outer(e, e) / _E
    c = np.cos(phase)
    s = -np.sin(phase)
    fwd = np.concatenate([c, s], axis=1).astype(np.float32)
    inv = (np.concatenate([c, s], axis=0) / _E).astype(np.float32)
    return fwd, inv


_FWD, _INV = _dft_mats()


def _dot(a, b):
    return jax.lax.dot_general(
        a, b, (((1,), (0,)), ((), ())),
        preferred_element_type=jnp.float32,
        precision=jax.lax.Precision.HIGHEST)


def _body(q_ref, k_ref, v_ref, fwd_ref, inv_ref, v_out_ref, corr_out_ref):
    q = q_ref[0, 0]
    k = k_ref[0, 0]
    v = v_ref[0, 0]
    fwd = fwd_ref[...]
    inv = inv_ref[...]
    r = q.shape[0]

    qkf = _dot(jnp.concatenate([q, k], axis=0), fwd)
    qr, qi = qkf[:r, :_E], qkf[:r, _E:]
    kr, ki = qkf[r:, :_E], qkf[r:, _E:]
    pr = qr * kr + qi * ki
    pi = qi * kr - qr * ki
    corr = _dot(jnp.concatenate([pr, pi], axis=1), inv)
    corr_out_ref[0] = corr.T

    neg = jnp.float32(-jnp.inf)
    m1 = jnp.max(corr, axis=1, keepdims=True)
    work = jnp.where(corr == m1, neg, corr)
    m2 = jnp.max(work, axis=1, keepdims=True)
    work = jnp.where(work == m2, neg, work)
    m3 = jnp.max(work, axis=1, keepdims=True)
    work = jnp.where(work == m3, neg, work)
    m4 = jnp.max(work, axis=1, keepdims=True)

    num = jnp.where(corr >= m4, jnp.exp(corr - m1), 0.0)
    denom = jnp.sum(num, axis=1, keepdims=True)
    w = num * (1.0 / denom)

    wvf = _dot(jnp.concatenate([w, v], axis=0), fwd)
    wr, wi = wvf[:r, :_E], wvf[:r, _E:]
    vr, vi = wvf[r:, :_E], wvf[r:, _E:]
    gr = wr * vr + wi * vi
    gi = wr * vi - wi * vr
    v_out_ref[0, 0] = _dot(jnp.concatenate([gr, gi], axis=1), inv)


def kernel(queries, keys, values):
    B, H, L, E = queries.shape
    lblk = 1024
    grid = (B, H, L // lblk)
    row_spec = pl.BlockSpec((1, 1, lblk, E), lambda b, h, l: (b, h, l, 0))
    out_v, out_corr = pl.pallas_call(
        _body,
        grid=grid,
        in_specs=[
            row_spec, row_spec, row_spec,
            pl.BlockSpec((_E, 2 * _E), lambda b, h, l: (0, 0)),
            pl.BlockSpec((2 * _E, _E), lambda b, h, l: (0, 0)),
        ],
        out_specs=[
            row_spec,
            pl.BlockSpec((1, E, lblk), lambda b, h, l: (b, 0, h * (L // lblk) + l)),
        ],
        out_shape=[
            jax.ShapeDtypeStruct((B, H, L, E), jnp.float32),
            jax.ShapeDtypeStruct((B, E, H * L), jnp.float32),
        ],
    )(queries, keys, values, jnp.asarray(_FWD), jnp.asarray(_INV))
    return (out_v, out_corr.reshape(B, E, H, L))


# speedup vs baseline: 1.1947x; 1.0544x over previous
import math

import numpy as np
import jax
import jax.numpy as jnp
from jax.experimental import pallas as pl

_E = 64
_TOPK = int(math.log(_E))


def _dft_mats():
    e = np.arange(_E)
    phase = 2.0 * np.pi * np.outer(e, e) / _E
    c = np.cos(phase)
    s = -np.sin(phase)
    fwd = np.concatenate([c, s], axis=1).astype(np.float32)
    inv = (np.concatenate([c, s], axis=0) / _E).astype(np.float32)
    return fwd, inv


_FWD, _INV = _dft_mats()


def _dot(a, b):
    return jax.lax.dot_general(
        a, b, (((1,), (0,)), ((), ())),
        preferred_element_type=jnp.float32,
        precision=jax.lax.Precision.HIGHEST)


def _body(q_ref, k_ref, v_ref, fwd_ref, inv_ref, v_out_ref, corr_out_ref):
    q = q_ref[0, 0]
    k = k_ref[0, 0]
    v = v_ref[0, 0]
    fwd = fwd_ref[...]
    inv = inv_ref[...]
    r = q.shape[0]

    qkf = _dot(jnp.concatenate([q, k], axis=0), fwd)
    qr, qi = qkf[:r, :_E], qkf[:r, _E:]
    kr, ki = qkf[r:, :_E], qkf[r:, _E:]
    pr = qr * kr + qi * ki
    pi = qi * kr - qr * ki
    corr = _dot(jnp.concatenate([pr, pi], axis=1), inv)
    corr_out_ref[0] = corr.T

    neg = jnp.float32(-jnp.inf)
    m1 = jnp.max(corr, axis=1, keepdims=True)
    work = jnp.where(corr == m1, neg, corr)
    m2 = jnp.max(work, axis=1, keepdims=True)
    work = jnp.where(work == m2, neg, work)
    m3 = jnp.max(work, axis=1, keepdims=True)
    work = jnp.where(work == m3, neg, work)
    m4 = jnp.max(work, axis=1, keepdims=True)

    num = jnp.where(corr >= m4, jnp.exp(corr - m1), 0.0)
    denom = jnp.sum(num, axis=1, keepdims=True)
    w = num * (1.0 / denom)

    wvf = _dot(jnp.concatenate([w, v], axis=0), fwd)
    wr, wi = wvf[:r, :_E], wvf[:r, _E:]
    vr, vi = wvf[r:, :_E], wvf[r:, _E:]
    gr = wr * vr + wi * vi
    gi = wr * vi - wi * vr
    v_out_ref[0, 0] = _dot(jnp.concatenate([gr, gi], axis=1), inv)


def kernel(queries, keys, values):
    B, H, L, E = queries.shape
    lblk = 2048
    grid = (B, H, L // lblk)
    row_spec = pl.BlockSpec((1, 1, lblk, E), lambda b, h, l: (b, h, l, 0))
    out_v, out_corr = pl.pallas_call(
        _body,
        grid=grid,
        in_specs=[
            row_spec, row_spec, row_spec,
            pl.BlockSpec((_E, 2 * _E), lambda b, h, l: (0, 0)),
            pl.BlockSpec((2 * _E, _E), lambda b, h, l: (0, 0)),
        ],
        out_specs=[
            row_spec,
            pl.BlockSpec((1, E, lblk), lambda b, h, l: (b, 0, h * (L // lblk) + l)),
        ],
        out_shape=[
            jax.ShapeDtypeStruct((B, H, L, E), jnp.float32),
            jax.ShapeDtypeStruct((B, E, H * L), jnp.float32),
        ],
    )(queries, keys, values, jnp.asarray(_FWD), jnp.asarray(_INV))
    return (out_v, out_corr.reshape(B, E, H, L))


# speedup vs baseline: 1.9370x; 1.6214x over previous
import math

import numpy as np
import jax
import jax.numpy as jnp
from jax.experimental import pallas as pl

_E = 64
_TOPK = int(math.log(_E))


def _dft_mats():
    e = np.arange(_E)
    phase = 2.0 * np.pi * np.outer(e, e) / _E
    c = np.cos(phase)
    s = -np.sin(phase)
    fwd = np.concatenate([c, s], axis=1)
    inv = np.concatenate([c, s], axis=0) / _E
    def split(m):
        hi = m.astype(np.float32).astype(jnp.bfloat16)
        lo = (m.astype(np.float32) - hi.astype(np.float32)).astype(jnp.bfloat16)
        return hi, lo
    return split(fwd), split(inv)


(_FWD_HI, _FWD_LO), (_INV_HI, _INV_LO) = _dft_mats()


def _dot(a, b):
    return jax.lax.dot_general(
        a, b, (((1,), (0,)), ((), ())),
        preferred_element_type=jnp.float32)


def _dot3(x, c_hi, c_lo):
    hi = x.astype(jnp.bfloat16)
    lo = (x - hi.astype(jnp.float32)).astype(jnp.bfloat16)
    return _dot(hi, c_hi) + (_dot(hi, c_lo) + _dot(lo, c_hi))


def _body(q_ref, k_ref, v_ref, fh_ref, fl_ref, ih_ref, il_ref,
          v_out_ref, corr_out_ref):
    q = q_ref[0, 0]
    k = k_ref[0, 0]
    v = v_ref[0, 0]
    f_hi, f_lo = fh_ref[...], fl_ref[...]
    i_hi, i_lo = ih_ref[...], il_ref[...]
    r = q.shape[0]

    qkf = _dot3(jnp.concatenate([q, k], axis=0), f_hi, f_lo)
    qr, qi = qkf[:r, :_E], qkf[:r, _E:]
    kr, ki = qkf[r:, :_E], qkf[r:, _E:]
    pr = qr * kr + qi * ki
    pi = qi * kr - qr * ki
    corr = _dot3(jnp.concatenate([pr, pi], axis=1), i_hi, i_lo)
    corr_out_ref[0] = corr.T

    neg = jnp.float32(-jnp.inf)
    m1 = jnp.max(corr, axis=1, keepdims=True)
    work = jnp.where(corr == m1, neg, corr)
    m2 = jnp.max(work, axis=1, keepdims=True)
    work = jnp.where(work == m2, neg, work)
    m3 = jnp.max(work, axis=1, keepdims=True)
    work = jnp.where(work == m3, neg, work)
    m4 = jnp.max(work, axis=1, keepdims=True)

    num = jnp.where(corr >= m4, jnp.exp(corr - m1), 0.0)
    denom = jnp.sum(num, axis=1, keepdims=True)
    w = num * (1.0 / denom)

    wvf = _dot(jnp.concatenate([w, v], axis=0).astype(jnp.bfloat16), f_hi)
    wr, wi = wvf[:r, :_E], wvf[:r, _E:]
    vr, vi = wvf[r:, :_E], wvf[r:, _E:]
    gr = wr * vr + wi * vi
    gi = wr * vi - wi * vr
    v_out_ref[0, 0] = _dot(
        jnp.concatenate([gr, gi], axis=1).astype(jnp.bfloat16), i_hi)


def kernel(queries, keys, values):
    B, H, L, E = queries.shape
    lblk = 2048
    grid = (B, H, L // lblk)
    row_spec = pl.BlockSpec((1, 1, lblk, E), lambda b, h, l: (b, h, l, 0))
    const_f = pl.BlockSpec((_E, 2 * _E), lambda b, h, l: (0, 0))
    const_i = pl.BlockSpec((2 * _E, _E), lambda b, h, l: (0, 0))
    out_v, out_corr = pl.pallas_call(
        _body,
        grid=grid,
        in_specs=[row_spec, row_spec, row_spec,
                  const_f, const_f, const_i, const_i],
        out_specs=[
            row_spec,
            pl.BlockSpec((1, E, lblk), lambda b, h, l: (b, 0, h * (L // lblk) + l)),
        ],
        out_shape=[
            jax.ShapeDtypeStruct((B, H, L, E), jnp.float32),
            jax.ShapeDtypeStruct((B, E, H * L), jnp.float32),
        ],
    )(queries, keys, values,
      jnp.asarray(_FWD_HI), jnp.asarray(_FWD_LO),
      jnp.asarray(_INV_HI), jnp.asarray(_INV_LO))
    return (out_v, out_corr.reshape(B, E, H, L))


# speedup vs baseline: 1.9938x; 1.0293x over previous
import math

import numpy as np
import jax
import jax.numpy as jnp
from jax.experimental import pallas as pl
from jax.experimental.pallas import tpu as pltpu

_E = 64
_TOPK = int(math.log(_E))


def _dft_mats():
    e = np.arange(_E)
    phase = 2.0 * np.pi * np.outer(e, e) / _E
    c = np.cos(phase)
    s = -np.sin(phase)
    fwd = np.concatenate([c, s], axis=1)
    inv = np.concatenate([c, s], axis=0) / _E
    def split(m):
        hi = m.astype(np.float32).astype(jnp.bfloat16)
        lo = (m.astype(np.float32) - hi.astype(np.float32)).astype(jnp.bfloat16)
        return hi, lo
    return split(fwd), split(inv)


(_FWD_HI, _FWD_LO), (_INV_HI, _INV_LO) = _dft_mats()


def _dot(a, b):
    return jax.lax.dot_general(
        a, b, (((1,), (0,)), ((), ())),
        preferred_element_type=jnp.float32)


def _dot3(x, c_hi, c_lo):
    hi = x.astype(jnp.bfloat16)
    lo = (x - hi.astype(jnp.float32)).astype(jnp.bfloat16)
    return _dot(hi, c_hi) + (_dot(hi, c_lo) + _dot(lo, c_hi))


def _body(q_ref, k_ref, v_ref, fh_ref, fl_ref, ih_ref, il_ref,
          v_out_ref, corr_out_ref):
    q = q_ref[0, 0]
    k = k_ref[0, 0]
    v = v_ref[0, 0]
    f_hi, f_lo = fh_ref[...], fl_ref[...]
    i_hi, i_lo = ih_ref[...], il_ref[...]
    r = q.shape[0]

    qkf = _dot3(jnp.concatenate([q, k], axis=0), f_hi, f_lo)
    qr, qi = qkf[:r, :_E], qkf[:r, _E:]
    kr, ki = qkf[r:, :_E], qkf[r:, _E:]
    pr = qr * kr + qi * ki
    pi = qi * kr - qr * ki
    corr = _dot3(jnp.concatenate([pr, pi], axis=1), i_hi, i_lo)
    corr_out_ref[0] = corr.T

    neg = jnp.float32(-jnp.inf)
    m1 = jnp.max(corr, axis=1, keepdims=True)
    work = jnp.where(corr == m1, neg, corr)
    m2 = jnp.max(work, axis=1, keepdims=True)
    work = jnp.where(work == m2, neg, work)
    m3 = jnp.max(work, axis=1, keepdims=True)
    work = jnp.where(work == m3, neg, work)
    m4 = jnp.max(work, axis=1, keepdims=True)

    num = jnp.where(corr >= m4, jnp.exp(corr - m1), 0.0)
    denom = jnp.sum(num, axis=1, keepdims=True)
    w = num * (1.0 / denom)

    wvf = _dot(jnp.concatenate([w, v], axis=0).astype(jnp.bfloat16), f_hi)
    wr, wi = wvf[:r, :_E], wvf[:r, _E:]
    vr, vi = wvf[r:, :_E], wvf[r:, _E:]
    gr = wr * vr + wi * vi
    gi = wr * vi - wi * vr
    v_out_ref[0, 0] = _dot(
        jnp.concatenate([gr, gi], axis=1).astype(jnp.bfloat16), i_hi)


def kernel(queries, keys, values):
    B, H, L, E = queries.shape
    lblk = 4096
    grid = (B, H, L // lblk)
    row_spec = pl.BlockSpec((1, 1, lblk, E), lambda b, h, l: (b, h, l, 0))
    const_f = pl.BlockSpec((_E, 2 * _E), lambda b, h, l: (0, 0))
    const_i = pl.BlockSpec((2 * _E, _E), lambda b, h, l: (0, 0))
    out_v, out_corr = pl.pallas_call(
        _body,
        grid=grid,
        in_specs=[row_spec, row_spec, row_spec,
                  const_f, const_f, const_i, const_i],
        out_specs=[
            row_spec,
            pl.BlockSpec((1, E, lblk), lambda b, h, l: (b, 0, h * (L // lblk) + l)),
        ],
        out_shape=[
            jax.ShapeDtypeStruct((B, H, L, E), jnp.float32),
            jax.ShapeDtypeStruct((B, E, H * L), jnp.float32),
        ],
        compiler_params=pltpu.CompilerParams(
            dimension_semantics=("parallel", "parallel", "parallel")),
    )(queries, keys, values,
      jnp.asarray(_FWD_HI), jnp.asarray(_FWD_LO),
      jnp.asarray(_INV_HI), jnp.asarray(_INV_LO))
    return (out_v, out_corr.reshape(B, E, H, L))


# speedup vs baseline: 3.0281x; 1.5188x over previous
import math

import numpy as np
import jax
import jax.numpy as jnp
from jax.experimental import pallas as pl
from jax.experimental.pallas import tpu as pltpu

_E = 64
_TOPK = int(math.log(_E))


def _dft_mats():
    e = np.arange(_E)
    phase = 2.0 * np.pi * np.outer(e, e) / _E
    c = np.cos(phase)
    s = -np.sin(phase)
    fwd = np.concatenate([c, s], axis=0)
    inv = np.concatenate([c, s], axis=1) / _E

    def split(m):
        hi = m.astype(np.float32).astype(jnp.bfloat16)
        lo = (m.astype(np.float32) - hi.astype(np.float32)).astype(jnp.bfloat16)
        return hi, lo

    return split(fwd), split(inv)


(_FWD_HI, _FWD_LO), (_INV_HI, _INV_LO) = _dft_mats()


def _dot(a, b):
    return jax.lax.dot_general(
        a, b, (((1,), (0,)), ((), ())),
        preferred_element_type=jnp.float32)


def _dot3(c_hi, c_lo, x):
    hi = x.astype(jnp.bfloat16)
    lo = (x - hi.astype(jnp.float32)).astype(jnp.bfloat16)
    return _dot(c_hi, hi) + (_dot(c_lo, hi) + _dot(c_hi, lo))


def _body(q_ref, k_ref, v_ref, fh_ref, fl_ref, ih_ref, il_ref,
          v_out_ref, corr_out_ref):
    r = q_ref.shape[2]
    qk_t = jnp.concatenate([q_ref[0, 0].T, k_ref[0, 0].T], axis=1)
    f_hi, f_lo = fh_ref[...], fl_ref[...]
    i_hi, i_lo = ih_ref[...], il_ref[...]

    qkf = _dot3(f_hi, f_lo, qk_t)
    qr, qi = qkf[:_E, :r], qkf[_E:, :r]
    kr, ki = qkf[:_E, r:], qkf[_E:, r:]
    pr = qr * kr + qi * ki
    pi = qi * kr - qr * ki
    corr = _dot3(i_hi, i_lo, jnp.concatenate([pr, pi], axis=0))
    corr_out_ref[0] = corr

    neg = jnp.float32(-jnp.inf)
    m1 = jnp.max(corr, axis=0, keepdims=True)
    work = jnp.where(corr == m1, neg, corr)
    m2 = jnp.max(work, axis=0, keepdims=True)
    work = jnp.where(work == m2, neg, work)
    m3 = jnp.max(work, axis=0, keepdims=True)
    work = jnp.where(work == m3, neg, work)
    m4 = jnp.max(work, axis=0, keepdims=True)

    num = jnp.where(corr >= m4, jnp.exp(corr - m1), 0.0)
    denom = jnp.sum(num, axis=0, keepdims=True)
    w = num * (1.0 / denom)

    wv_t = jnp.concatenate([w, v_ref[0, 0].T], axis=1).astype(jnp.bfloat16)
    wvf = _dot(f_hi, wv_t)
    wr, wi = wvf[:_E, :r], wvf[_E:, :r]
    vr, vi = wvf[:_E, r:], wvf[_E:, r:]
    gr = wr * vr + wi * vi
    gi = wr * vi - wi * vr
    agg = _dot(i_hi, jnp.concatenate([gr, gi], axis=0).astype(jnp.bfloat16))
    v_out_ref[0, 0] = agg.T


def kernel(queries, keys, values):
    B, H, L, E = queries.shape
    lblk = 4096
    grid = (B, H, L // lblk)
    row_spec = pl.BlockSpec((1, 1, lblk, E), lambda b, h, l: (b, h, l, 0))
    const_f = pl.BlockSpec((2 * _E, _E), lambda b, h, l: (0, 0))
    const_i = pl.BlockSpec((_E, 2 * _E), lambda b, h, l: (0, 0))
    out_v, out_corr = pl.pallas_call(
        _body,
        grid=grid,
        in_specs=[row_spec, row_spec, row_spec,
                  const_f, const_f, const_i, const_i],
        out_specs=[
            row_spec,
            pl.BlockSpec((1, E, lblk), lambda b, h, l: (b, 0, h * (L // lblk) + l)),
        ],
        out_shape=[
            jax.ShapeDtypeStruct((B, H, L, E), jnp.float32),
            jax.ShapeDtypeStruct((B, E, H * L), jnp.float32),
        ],
        compiler_params=pltpu.CompilerParams(
            dimension_semantics=("parallel", "parallel", "parallel")),
    )(queries, keys, values,
      jnp.asarray(_FWD_HI), jnp.asarray(_FWD_LO),
      jnp.asarray(_INV_HI), jnp.asarray(_INV_LO))
    return (out_v, out_corr.reshape(B, E, H, L))


# speedup vs baseline: 3.2213x; 1.0638x over previous
import math

import numpy as np
import jax
import jax.numpy as jnp
from jax.experimental import pallas as pl
from jax.experimental.pallas import tpu as pltpu

_E = 64
_TOPK = int(math.log(_E))


def _dft_mats():
    e = np.arange(_E)
    phase = 2.0 * np.pi * np.outer(e, e) / _E
    c = np.cos(phase)
    s = -np.sin(phase)
    fwd = np.concatenate([c, s], axis=0)
    inv = np.concatenate([c, s], axis=1) / _E

    def split(m):
        hi = m.astype(np.float32).astype(jnp.bfloat16)
        lo = (m.astype(np.float32) - hi.astype(np.float32)).astype(jnp.bfloat16)
        return hi, lo

    return split(fwd), split(inv)


(_FWD_HI, _FWD_LO), (_INV_HI, _INV_LO) = _dft_mats()


def _dot(a, b):
    return jax.lax.dot_general(
        a, b, (((1,), (0,)), ((), ())),
        preferred_element_type=jnp.float32)


def _dot3(c_hi, c_lo, x):
    hi = x.astype(jnp.bfloat16)
    lo = (x - hi.astype(jnp.float32)).astype(jnp.bfloat16)
    return _dot(c_hi, hi) + (_dot(c_lo, hi) + _dot(c_hi, lo))


def _body(q_ref, k_ref, v_ref, fh_ref, fl_ref, ih_ref, il_ref,
          v_out_ref, corr_out_ref):
    hg, lblk = q_ref.shape[1], q_ref.shape[2]
    r = hg * lblk
    q = q_ref[0].reshape(r, _E)
    k = k_ref[0].reshape(r, _E)
    qk_t = jnp.concatenate([q.T, k.T], axis=1)
    f_hi, f_lo = fh_ref[...], fl_ref[...]
    i_hi, i_lo = ih_ref[...], il_ref[...]

    qkf = _dot3(f_hi, f_lo, qk_t)
    qr, qi = qkf[:_E, :r], qkf[_E:, :r]
    kr, ki = qkf[:_E, r:], qkf[_E:, r:]
    pr = qr * kr + qi * ki
    pi = qi * kr - qr * ki
    corr = _dot3(i_hi, i_lo, jnp.concatenate([pr, pi], axis=0))
    corr_out_ref[0] = corr.reshape(_E, hg, lblk)

    neg = jnp.float32(-jnp.inf)
    m1 = jnp.max(corr, axis=0, keepdims=True)
    work = jnp.where(corr == m1, neg, corr)
    m2 = jnp.max(work, axis=0, keepdims=True)
    work = jnp.where(work == m2, neg, work)
    m3 = jnp.max(work, axis=0, keepdims=True)
    work = jnp.where(work == m3, neg, work)
    m4 = jnp.max(work, axis=0, keepdims=True)

    num = jnp.where(corr >= m4, jnp.exp(corr - m1), 0.0)
    denom = jnp.sum(num, axis=0, keepdims=True)
    w = num * (1.0 / denom)

    wv_t = jnp.concatenate(
        [w, v_ref[0].reshape(r, _E).T], axis=1).astype(jnp.bfloat16)
    wvf = _dot(f_hi, wv_t)
    wr, wi = wvf[:_E, :r], wvf[_E:, :r]
    vr, vi = wvf[:_E, r:], wvf[_E:, r:]
    gr = wr * vr + wi * vi
    gi = wr * vi - wi * vr
    agg = _dot(i_hi, jnp.concatenate([gr, gi], axis=0).astype(jnp.bfloat16))
    v_out_ref[0] = agg.T.reshape(hg, lblk, _E)


def kernel(queries, keys, values):
    B, H, L, E = queries.shape
    hg, lblk = 8, 512
    grid = (B, H // hg, L // lblk)
    row_spec = pl.BlockSpec((1, hg, lblk, E), lambda b, h, l: (b, h, l, 0))
    const_f = pl.BlockSpec((2 * _E, _E), lambda b, h, l: (0, 0))
    const_i = pl.BlockSpec((_E, 2 * _E), lambda b, h, l: (0, 0))
    out_v, out_corr = pl.pallas_call(
        _body,
        grid=grid,
        in_specs=[row_spec, row_spec, row_spec,
                  const_f, const_f, const_i, const_i],
        out_specs=[
            row_spec,
            pl.BlockSpec((1, E, hg, lblk), lambda b, h, l: (b, 0, h, l)),
        ],
        out_shape=[
            jax.ShapeDtypeStruct((B, H, L, E), jnp.float32),
            jax.ShapeDtypeStruct((B, E, H, L), jnp.float32),
        ],
        compiler_params=pltpu.CompilerParams(
            dimension_semantics=("parallel", "parallel", "parallel")),
    )(queries, keys, values,
      jnp.asarray(_FWD_HI), jnp.asarray(_FWD_LO),
      jnp.asarray(_INV_HI), jnp.asarray(_INV_LO))
    return (out_v, out_corr)


# speedup vs baseline: 3.3707x; 1.0464x over previous
import math

import numpy as np
import jax
import jax.numpy as jnp
from jax.experimental import pallas as pl
from jax.experimental.pallas import tpu as pltpu

_E = 64
_TOPK = int(math.log(_E))


def _dft_mats():
    e = np.arange(_E)
    phase = 2.0 * np.pi * np.outer(e, e) / _E
    c = np.cos(phase)
    s = -np.sin(phase)
    fwd = np.concatenate([c, s], axis=0)
    inv = np.concatenate([c, s], axis=1) / _E

    def split(m):
        hi = m.astype(np.float32).astype(jnp.bfloat16)
        lo = (m.astype(np.float32) - hi.astype(np.float32)).astype(jnp.bfloat16)
        return hi, lo

    return split(fwd), split(inv)


(_FWD_HI, _FWD_LO), (_INV_HI, _INV_LO) = _dft_mats()


def _dot(a, b):
    return jax.lax.dot_general(
        a, b, (((1,), (0,)), ((), ())),
        preferred_element_type=jnp.float32)


def _dot3(c_hi, c_lo, x):
    hi = x.astype(jnp.bfloat16)
    lo = (x - hi.astype(jnp.float32)).astype(jnp.bfloat16)
    return _dot(c_hi, hi) + (_dot(c_lo, hi) + _dot(c_hi, lo))


def _body(q_ref, k_ref, v_ref, fh_ref, fl_ref, ih_ref, il_ref,
          v_out_ref, corr_out_ref):
    hg, lblk = q_ref.shape[1], q_ref.shape[2]
    r = hg * lblk
    f_hi, f_lo = fh_ref[...], fl_ref[...]
    i_hi, i_lo = ih_ref[...], il_ref[...]

    qf = _dot3(f_hi, f_lo, q_ref[0].reshape(r, _E).T)
    kf = _dot3(f_hi, f_lo, k_ref[0].reshape(r, _E).T)
    qr, qi = qf[:_E], qf[_E:]
    kr, ki = kf[:_E], kf[_E:]
    pr = qr * kr + qi * ki
    pi = qi * kr - qr * ki
    corr = _dot3(i_hi, i_lo, jnp.concatenate([pr, pi], axis=0))
    corr_out_ref[0] = corr.reshape(_E, hg, lblk)

    neg = jnp.float32(-jnp.inf)
    m1 = jnp.max(corr, axis=0, keepdims=True)
    work = jnp.where(corr == m1, neg, corr)
    m2 = jnp.max(work, axis=0, keepdims=True)
    work = jnp.where(work == m2, neg, work)
    m3 = jnp.max(work, axis=0, keepdims=True)
    work = jnp.where(work == m3, neg, work)
    m4 = jnp.max(work, axis=0, keepdims=True)

    num = jnp.where(corr >= m4, jnp.exp(corr - m1), 0.0)
    denom = jnp.sum(num, axis=0, keepdims=True)
    w = num * (1.0 / denom)

    wf = _dot(f_hi, w.astype(jnp.bfloat16))
    vf = _dot(f_hi, v_ref[0].reshape(r, _E).T.astype(jnp.bfloat16))
    wr, wi = wf[:_E], wf[_E:]
    vr, vi = vf[:_E], vf[_E:]
    gr = wr * vr + wi * vi
    gi = wr * vi - wi * vr
    agg = _dot(i_hi, jnp.concatenate([gr, gi], axis=0).astype(jnp.bfloat16))
    v_out_ref[0] = agg.T.reshape(hg, lblk, _E)


def kernel(queries, keys, values):
    B, H, L, E = queries.shape
    hg, lblk = 8, 1024
    grid = (B, H // hg, L // lblk)
    row_spec = pl.BlockSpec((1, hg, lblk, E), lambda b, h, l: (b, h, l, 0))
    const_f = pl.BlockSpec((2 * _E, _E), lambda b, h, l: (0, 0))
    const_i = pl.BlockSpec((_E, 2 * _E), lambda b, h, l: (0, 0))
    out_v, out_corr = pl.pallas_call(
        _body,
        grid=grid,
        in_specs=[row_spec, row_spec, row_spec,
                  const_f, const_f, const_i, const_i],
        out_specs=[
            row_spec,
            pl.BlockSpec((1, E, hg, lblk), lambda b, h, l: (b, 0, h, l)),
        ],
        out_shape=[
            jax.ShapeDtypeStruct((B, H, L, E), jnp.float32),
            jax.ShapeDtypeStruct((B, E, H, L), jnp.float32),
        ],
        compiler_params=pltpu.CompilerParams(
            dimension_semantics=("parallel", "parallel", "parallel")),
    )(queries, keys, values,
      jnp.asarray(_FWD_HI), jnp.asarray(_FWD_LO),
      jnp.asarray(_INV_HI), jnp.asarray(_INV_LO))
    return (out_v, out_corr)


# speedup vs baseline: 3.5744x; 1.0604x over previous
import math

import numpy as np
import jax
import jax.numpy as jnp
from jax.experimental import pallas as pl
from jax.experimental.pallas import tpu as pltpu

_E = 64
_TOPK = int(math.log(_E))


def _dft_mats():
    e = np.arange(_E)
    phase = 2.0 * np.pi * np.outer(e, e) / _E
    c = np.cos(phase)
    s = -np.sin(phase)
    fwd = np.concatenate([c, s], axis=0)
    inv = np.concatenate([c, s], axis=1) / _E

    def split(m):
        hi = m.astype(np.float32).astype(jnp.bfloat16)
        lo = (m.astype(np.float32) - hi.astype(np.float32)).astype(jnp.bfloat16)
        return hi, lo

    return split(fwd), split(inv)


(_FWD_HI, _FWD_LO), (_INV_HI, _INV_LO) = _dft_mats()


def _dot(a, b):
    return jax.lax.dot_general(
        a, b, (((1,), (0,)), ((), ())),
        preferred_element_type=jnp.float32)


def _dot_t(a, b):
    return jax.lax.dot_general(
        a, b, (((1,), (1,)), ((), ())),
        preferred_element_type=jnp.float32)


def _dot3(c_hi, c_lo, x):
    hi = x.astype(jnp.bfloat16)
    lo = (x - hi.astype(jnp.float32)).astype(jnp.bfloat16)
    return _dot(c_hi, hi) + (_dot(c_lo, hi) + _dot(c_hi, lo))


def _dot3_t(c_hi, c_lo, x):
    hi = x.astype(jnp.bfloat16)
    lo = (x - hi.astype(jnp.float32)).astype(jnp.bfloat16)
    return _dot_t(c_hi, hi) + (_dot_t(c_lo, hi) + _dot_t(c_hi, lo))


def _body(q_ref, k_ref, v_ref, fh_ref, fl_ref, ih_ref, il_ref,
          v_out_ref, corr_out_ref):
    hg, lblk = q_ref.shape[1], q_ref.shape[2]
    r = hg * lblk
    f_hi, f_lo = fh_ref[...], fl_ref[...]
    i_hi, i_lo = ih_ref[...], il_ref[...]

    qf = _dot3_t(f_hi, f_lo, q_ref[0].reshape(r, _E))
    kf = _dot3_t(f_hi, f_lo, k_ref[0].reshape(r, _E))
    qr, qi = qf[:_E], qf[_E:]
    kr, ki = kf[:_E], kf[_E:]
    pr = qr * kr + qi * ki
    pi = qi * kr - qr * ki
    corr = _dot3(i_hi, i_lo, jnp.concatenate([pr, pi], axis=0))
    corr_out_ref[0] = corr.reshape(_E, hg, lblk)

    neg = jnp.float32(-jnp.inf)
    m1 = jnp.max(corr, axis=0, keepdims=True)
    work = jnp.where(corr == m1, neg, corr)
    m2 = jnp.max(work, axis=0, keepdims=True)
    work = jnp.where(work == m2, neg, work)
    m3 = jnp.max(work, axis=0, keepdims=True)
    work = jnp.where(work == m3, neg, work)
    m4 = jnp.max(work, axis=0, keepdims=True)

    num = jnp.where(corr >= m4, jnp.exp(corr - m1), 0.0)
    denom = jnp.sum(num, axis=0, keepdims=True)
    w = num * (1.0 / denom)

    wf = _dot(f_hi, w.astype(jnp.bfloat16))
    vf = _dot_t(f_hi, v_ref[0].reshape(r, _E).astype(jnp.bfloat16))
    wr, wi = wf[:_E], wf[_E:]
    vr, vi = vf[:_E], vf[_E:]
    gr = wr * vr + wi * vi
    gi = wr * vi - wi * vr
    agg = jax.lax.dot_general(
        jnp.concatenate([gr, gi], axis=0).astype(jnp.bfloat16), i_hi,
        (((0,), (1,)), ((), ())), preferred_element_type=jnp.float32)
    v_out_ref[0] = agg.reshape(hg, lblk, _E)


def kernel(queries, keys, values):
    B, H, L, E = queries.shape
    hg, lblk = 8, 1024
    grid = (B, H // hg, L // lblk)
    row_spec = pl.BlockSpec((1, hg, lblk, E), lambda b, h, l: (b, h, l, 0))
    const_f = pl.BlockSpec((2 * _E, _E), lambda b, h, l: (0, 0))
    const_i = pl.BlockSpec((_E, 2 * _E), lambda b, h, l: (0, 0))
    out_v, out_corr = pl.pallas_call(
        _body,
        grid=grid,
        in_specs=[row_spec, row_spec, row_spec,
                  const_f, const_f, const_i, const_i],
        out_specs=[
            row_spec,
            pl.BlockSpec((1, E, hg, lblk), lambda b, h, l: (b, 0, h, l)),
        ],
        out_shape=[
            jax.ShapeDtypeStruct((B, H, L, E), jnp.float32),
            jax.ShapeDtypeStruct((B, E, H, L), jnp.float32),
        ],
        compiler_params=pltpu.CompilerParams(
            dimension_semantics=("parallel", "parallel", "parallel")),
    )(queries, keys, values,
      jnp.asarray(_FWD_HI), jnp.asarray(_FWD_LO),
      jnp.asarray(_INV_HI), jnp.asarray(_INV_LO))
    return (out_v, out_corr)
